# Initial kernel scaffold; baseline (speedup 1.0000x reference)
import jax
import jax.numpy as jnp
from jax.experimental import pallas as pl


def kernel(node_embed, edge_index, edge_attr, We, be, W1, b1, gamma, beta, W2, b2, eps_param):
    raise NotImplementedError("write your pallas kernel here")


# speedup vs baseline: 1.6581x; 1.6581x over previous
import functools

import jax
import jax.numpy as jnp
from jax import lax
from jax.experimental import pallas as pl
from jax.experimental.pallas import tpu as pltpu
from jax.experimental.pallas import tpu_sc as plsc

N = 10000
E = 320000
D = 128

N_TILES = 32
CHUNK = 128
E_PER_TILE = 10240
E_PAD = N_TILES * E_PER_TILE
N_CHUNKS = E_PER_TILE // CHUNK
N_ACC = 10240
ROWS_PER_TILE = N_ACC // 16
PAD_DST = N_ACC - 1


def _emb_body(a_ref, we_ref, be_ref, out_ref):
    out_ref[...] = (
        jnp.dot(a_ref[...], we_ref[...], preferred_element_type=jnp.float32)
        + be_ref[...]
    )


def _edge_encoder(edge_attr_pad, We_pad, be):
    blk = 1024
    grid = E_PAD // blk
    return pl.pallas_call(
        _emb_body,
        grid=(grid,),
        in_specs=[
            pl.BlockSpec((blk, 8), lambda i: (i, 0)),
            pl.BlockSpec((8, D), lambda i: (0, 0)),
            pl.BlockSpec((1, D), lambda i: (0, 0)),
        ],
        out_specs=pl.BlockSpec((blk, D), lambda i: (i, 0)),
        out_shape=jax.ShapeDtypeStruct((E_PAD, D), jnp.float32),
    )(edge_attr_pad, We_pad, be.reshape(1, D))


def _sc_body(node_hbm, emb_hbm, src_hbm, dst_hbm, out_hbm,
             acc, srcv, dstv, xjv, embv, sem):
    cid = lax.axis_index("c")
    sid = lax.axis_index("s")
    wid = cid * 16 + sid

    def _zero_row(r, _):
        for d in range(8):
            embv[r, pl.ds(16 * d, 16)] = jnp.zeros((16,), jnp.float32)
        return 0

    lax.fori_loop(0, CHUNK, _zero_row, 0)
    for b in range(ROWS_PER_TILE // CHUNK):
        pltpu.sync_copy(embv, acc.at[pl.ds(sid * ROWS_PER_TILE + b * CHUNK, CHUNK)])
    plsc.subcore_barrier()

    base = wid * E_PER_TILE

    def _chunk(g, _):
        off = pl.multiple_of(base + g * CHUNK, CHUNK)
        pltpu.sync_copy(src_hbm.at[pl.ds(off, CHUNK)], srcv)
        pltpu.sync_copy(dst_hbm.at[pl.ds(off, CHUNK)], dstv)
        pltpu.async_copy(node_hbm.at[srcv], xjv, sem).wait()
        pltpu.sync_copy(emb_hbm.at[pl.ds(off, CHUNK)], embv)

        def _edge(e, _):
            for d in range(8):
                s = pl.ds(16 * d, 16)
                embv[e, s] = jnp.maximum(embv[e, s] + xjv[e, s], 0.0)
            return 0

        lax.fori_loop(0, CHUNK, _edge, 0)
        pltpu.sync_copy(embv, acc.at[dstv], add=True)
        return 0

    lax.fori_loop(0, N_CHUNKS, _chunk, 0)
    plsc.subcore_barrier()

    for b in range(ROWS_PER_TILE // CHUNK):
        r0 = sid * ROWS_PER_TILE + b * CHUNK
        pltpu.sync_copy(acc.at[pl.ds(r0, CHUNK)], out_hbm.at[cid, pl.ds(r0, CHUNK)])


def _sc_scatter(node_embed, emb, src, dst):
    mesh = plsc.VectorSubcoreMesh(core_axis_name="c", subcore_axis_name="s")
    f = pl.kernel(
        _sc_body,
        out_type=jax.ShapeDtypeStruct((2, N_ACC, D), jnp.float32),
        mesh=mesh,
        scratch_types=[
            pltpu.VMEM_SHARED((N_ACC, D), jnp.float32),
            pltpu.VMEM((CHUNK,), jnp.int32),
            pltpu.VMEM((CHUNK,), jnp.int32),
            pltpu.VMEM((CHUNK, D), jnp.float32),
            pltpu.VMEM((CHUNK, D), jnp.float32),
            pltpu.SemaphoreType.DMA,
        ],
    )
    return f(node_embed, emb, src, dst)


def _mlp_body(x_ref, v_ref, eps_ref, w1_ref, b1_ref, g_ref, bt_ref, w2_ref,
              b2_ref, out_ref):
    x = x_ref[...]
    v = v_ref[0, :N, :] + v_ref[1, :N, :]
    h = x * (1.0 + eps_ref[...]) + v
    z1 = jnp.dot(h, w1_ref[...], preferred_element_type=jnp.float32) + b1_ref[...]
    mean = jnp.mean(z1, axis=0, keepdims=True)
    var = jnp.mean((z1 - mean) ** 2, axis=0, keepdims=True)
    z1n = g_ref[...] * ((z1 - mean) / jnp.sqrt(var + 1e-5)) + bt_ref[...]
    z2 = jnp.maximum(z1n, 0.0)
    out_ref[...] = (
        jnp.dot(z2, w2_ref[...], preferred_element_type=jnp.float32) + b2_ref[...]
    )


def _mlp(node_embed, v, eps_param, W1, b1, gamma, beta, W2, b2):
    return pl.pallas_call(
        _mlp_body,
        out_shape=jax.ShapeDtypeStruct((N, D), jnp.float32),
    )(
        node_embed,
        v,
        eps_param.reshape(1, 1),
        W1,
        b1.reshape(1, 2 * D),
        gamma.reshape(1, 2 * D),
        beta.reshape(1, 2 * D),
        W2,
        b2.reshape(1, D),
    )


def kernel(node_embed, edge_index, edge_attr, We, be, W1, b1, gamma, beta, W2,
           b2, eps_param):
    src = jnp.pad(edge_index[1].astype(jnp.int32), (0, E_PAD - E))
    dst = jnp.pad(edge_index[0].astype(jnp.int32), (0, E_PAD - E),
                  constant_values=PAD_DST)
    ea = jnp.pad(edge_attr, ((0, E_PAD - E), (0, 1)))
    We_pad = jnp.pad(We, ((0, 1), (0, 0)))

    emb = _edge_encoder(ea, We_pad, be)
    v = _sc_scatter(node_embed, emb, src, dst)
    return _mlp(node_embed, v, eps_param, W1, b1, gamma, beta, W2, b2)
